# baseline (speedup 1.0000x reference)
import functools

import jax
import jax.numpy as jnp
from jax import lax
from jax.experimental import pallas as pl
from jax.experimental.pallas import tpu as pltpu
from jax.experimental.pallas import tpu_sc as plsc

DIM = 64
B = 16384
NEG = 20
NC, NS = 2, 16
NW = NC * NS
NB = B // NW
NNEG = NB * NEG
NLOOK = NB + NNEG
CHUNK = 256
NCHUNK = NLOOK // CHUNK
NBUF = 5
NROUND = (NCHUNK + NBUF - 1) // NBUF
GR = 128


def _dot16(a_ref, a_rows, b_ref, b_rows):
    dvec = jnp.zeros((16,), jnp.int32)
    accs = [jnp.zeros((16,), jnp.float32) for _ in range(4)]
    for d in range(DIM):
        accs[d % 4] = accs[d % 4] + (plsc.load_gather(a_ref, [a_rows, dvec])
                                     * plsc.load_gather(b_ref, [b_rows, dvec]))
        dvec = dvec + 1
    return (accs[0] + accs[1]) + (accs[2] + accs[3])


def _sc_body(center_hbm, context_hbm, negflat_hbm, embc_hbm, embx_hbm,
             pos_out, neg_out,
             idx_c, idx_k, vc, bufs, chunk_s, sems, semP):
    wid = lax.axis_index("s") * NC + lax.axis_index("c")
    base = wid * NB
    nbase = wid * NNEG

    pltpu.sync_copy(center_hbm.at[pl.ds(base, NB)], idx_c)
    pltpu.sync_copy(context_hbm.at[pl.ds(base, NB)], idx_k.at[pl.ds(0, NB)])
    pltpu.sync_copy(negflat_hbm.at[pl.ds(nbase, NNEG)], idx_k.at[pl.ds(NB, NNEG)])

    def dbl_c(t, carry):
        for u in range(4):
            sl = pl.ds(t * 64 + u * 16, 16)
            idx_c[sl] = idx_c[sl] * 2
        return carry
    lax.fori_loop(0, NB // 64, dbl_c, 0)

    def dbl_k(t, carry):
        for u in range(4):
            sl = pl.ds(t * 64 + u * 16, 16)
            idx_k[sl] = idx_k[sl] * 2
        return carry
    lax.fori_loop(0, NLOOK // 64, dbl_k, 0)

    vc_copies = []
    for t in range(NB // GR):
        sl = pl.ds(t * GR, GR)
        vc_copies.append(pltpu.async_copy(embc_hbm.at[idx_c.at[sl]], vc.at[sl], semP))

    def start_chunk(ci, buf, sem):
        off = ci * CHUNK
        for o in (0, GR):
            pltpu.async_copy(embx_hbm.at[idx_k.at[pl.ds(off + o, GR)]],
                             buf.at[pl.ds(o, GR)], sem)

    def wait_chunk(buf, sem):
        pltpu.make_async_copy(embx_hbm.at[pl.ds(0, CHUNK)], buf, sem).wait()

    for b in range(NBUF - 1):
        start_chunk(b, bufs[b], sems[b])

    for c in vc_copies:
        c.wait()

    lane = lax.iota(jnp.int32, 16)

    def compute_chunk(ci, buf, sbuf):
        def g_body(g, carry):
            rvec = g * 16 + lane
            rgl = ci * CHUNK + rvec
            brow = jnp.where(rgl < NB, rgl, (rgl - NB) // NEG)
            sbuf[pl.ds(g * 16, 16)] = _dot16(buf, rvec, vc, brow)
            return carry
        lax.fori_loop(0, CHUNK // 16, g_body, 0)

    def flush_scores(ci, sbuf):
        @pl.when(ci < NB // CHUNK)
        def _():
            pltpu.sync_copy(sbuf, pos_out.at[pl.ds(base + ci * CHUNK, CHUNK)])

        @pl.when(ci >= NB // CHUNK)
        def _():
            pltpu.sync_copy(
                sbuf, neg_out.at[pl.ds(nbase + (ci - NB // CHUNK) * CHUNK, CHUNK)])

    def round_body(r, carry):
        for b in range(NBUF):
            ci = r * NBUF + b

            @pl.when(ci < NCHUNK)
            def _():
                wait_chunk(bufs[b], sems[b])

                @pl.when(ci + NBUF - 1 < NCHUNK)
                def _():
                    start_chunk(ci + NBUF - 1, bufs[b - 1], sems[b - 1])
                compute_chunk(ci, bufs[b], chunk_s)
                flush_scores(ci, chunk_s)
        return carry

    lax.fori_loop(0, NROUND, round_body, 0)


def _sc_entry(center_hbm, context_hbm, negflat_hbm, embc_hbm, embx_hbm,
              pos_out, neg_out,
              idx_c, idx_k, vc, b0, b1, b2, b3, b4, chunk_s,
              s0, s1, s2, s3, s4, semP):
    _sc_body(center_hbm, context_hbm, negflat_hbm, embc_hbm, embx_hbm,
             pos_out, neg_out, idx_c, idx_k, vc,
             [b0, b1, b2, b3, b4], chunk_s,
             [s0, s1, s2, s3, s4], semP)


_sc_scores = pl.kernel(
    _sc_entry,
    out_type=(jax.ShapeDtypeStruct((B,), jnp.float32),
              jax.ShapeDtypeStruct((B * NEG,), jnp.float32)),
    mesh=plsc.VectorSubcoreMesh(core_axis_name="c", subcore_axis_name="s"),
    scratch_types=[
        pltpu.VMEM((NB,), jnp.int32),
        pltpu.VMEM((NLOOK,), jnp.int32),
        pltpu.VMEM((NB, DIM), jnp.float32),
        pltpu.VMEM((CHUNK, DIM), jnp.float32),
        pltpu.VMEM((CHUNK, DIM), jnp.float32),
        pltpu.VMEM((CHUNK, DIM), jnp.float32),
        pltpu.VMEM((CHUNK, DIM), jnp.float32),
        pltpu.VMEM((CHUNK, DIM), jnp.float32),
        pltpu.VMEM((CHUNK,), jnp.float32),
        pltpu.SemaphoreType.DMA,
        pltpu.SemaphoreType.DMA,
        pltpu.SemaphoreType.DMA,
        pltpu.SemaphoreType.DMA,
        pltpu.SemaphoreType.DMA,
        pltpu.SemaphoreType.DMA,
    ],
    compiler_params=pltpu.CompilerParams(needs_layout_passes=False,
                                         use_tc_tiling_on_sc=False),
)


VOCAB = 1000000
TRB = 1024


def _tr_body(in_ref, out_ref):
    out_ref[:, 0:DIM] = in_ref[...].T


_tc_transpose = pl.pallas_call(
    _tr_body,
    grid=((VOCAB + TRB - 1) // TRB,),
    in_specs=[pl.BlockSpec((DIM, TRB), lambda i: (0, i))],
    out_specs=pl.BlockSpec((TRB, 128), lambda i: (i, 0)),
    out_shape=jax.ShapeDtypeStruct((VOCAB, 128), jnp.float32),
)


def _to_rows(emb):
    return _tc_transpose(emb.T).reshape(2 * VOCAB, DIM)


def _logsig(x):
    return jnp.minimum(x, 0.0) - jnp.log1p(jnp.exp(-jnp.abs(x)))


def _loss_body(pos_ref, neg_ref, out_ref):
    loss = -jnp.sum(_logsig(pos_ref[...])) - jnp.sum(_logsig(-neg_ref[...]))
    out_ref[0, 0] = loss


_tc_loss = pl.pallas_call(
    _loss_body,
    out_shape=jax.ShapeDtypeStruct((1, 1), jnp.float32),
    out_specs=pl.BlockSpec(memory_space=pltpu.SMEM),
)


def kernel(center_word, context_word, negative_samples, emb_center, emb_context):
    neg_flat = negative_samples.reshape(-1)
    pos_s, neg_s = _sc_scores(center_word, context_word, neg_flat,
                              _to_rows(emb_center), _to_rows(emb_context))
    loss = _tc_loss(pos_s.reshape(B // 128, 128), neg_s.reshape(B * NEG // 128, 128))
    return loss[0, 0]


# speedup vs baseline: 1.3701x; 1.3701x over previous
import functools

import jax
import jax.numpy as jnp
from jax import lax
from jax.experimental import pallas as pl
from jax.experimental.pallas import tpu as pltpu
from jax.experimental.pallas import tpu_sc as plsc

DIM = 64
B = 16384
NEG = 20
NC, NS = 2, 16
NW = NC * NS
NB = B // NW
NNEG = NB * NEG
NLOOK = NB + NNEG
CHUNK = 256
NCHUNK = NLOOK // CHUNK
NBUF = 5
NROUND = (NCHUNK + NBUF - 1) // NBUF
GR = 128


def _dot16(a_ref, a_rows, b_ref, b_rows):
    dvec = jnp.zeros((16,), jnp.int32)
    accs = [jnp.zeros((16,), jnp.float32) for _ in range(4)]
    for d in range(DIM):
        accs[d % 4] = accs[d % 4] + (plsc.load_gather(a_ref, [a_rows, dvec])
                                     * plsc.load_gather(b_ref, [b_rows, dvec]))
        dvec = dvec + 1
    return (accs[0] + accs[1]) + (accs[2] + accs[3])


def _sc_body(center_hbm, context_hbm, negflat_hbm, tbl_hbm,
             pos_out, neg_out,
             idx_c, idx_k, vc, bufs, chunk_s, sems, semP):
    wid = lax.axis_index("s") * NC + lax.axis_index("c")
    base = wid * NB
    nbase = wid * NNEG

    pltpu.sync_copy(center_hbm.at[pl.ds(base, NB)], idx_c)
    pltpu.sync_copy(context_hbm.at[pl.ds(base, NB)], idx_k.at[pl.ds(0, NB)])
    pltpu.sync_copy(negflat_hbm.at[pl.ds(nbase, NNEG)], idx_k.at[pl.ds(NB, NNEG)])

    def dbl_c(t, carry):
        for u in range(4):
            sl = pl.ds(t * 64 + u * 16, 16)
            idx_c[sl] = idx_c[sl] * 2
        return carry
    lax.fori_loop(0, NB // 64, dbl_c, 0)

    def dbl_k(t, carry):
        for u in range(4):
            sl = pl.ds(t * 64 + u * 16, 16)
            idx_k[sl] = idx_k[sl] * 2 + 1
        return carry
    lax.fori_loop(0, NLOOK // 64, dbl_k, 0)

    vc_copies = []
    for t in range(NB // GR):
        sl = pl.ds(t * GR, GR)
        vc_copies.append(pltpu.async_copy(tbl_hbm.at[idx_c.at[sl]], vc.at[sl], semP))

    def start_chunk(ci, buf, sem):
        off = ci * CHUNK
        for o in (0, GR):
            pltpu.async_copy(tbl_hbm.at[idx_k.at[pl.ds(off + o, GR)]],
                             buf.at[pl.ds(o, GR)], sem)

    def wait_chunk(buf, sem):
        pltpu.make_async_copy(tbl_hbm.at[pl.ds(0, CHUNK)], buf, sem).wait()

    for b in range(NBUF - 1):
        start_chunk(b, bufs[b], sems[b])

    for c in vc_copies:
        c.wait()

    lane = lax.iota(jnp.int32, 16)

    def compute_chunk(ci, buf, sbuf):
        def g_body(g, carry):
            rvec = g * 16 + lane
            rgl = ci * CHUNK + rvec
            brow = jnp.where(rgl < NB, rgl, (rgl - NB) // NEG)
            sbuf[pl.ds(g * 16, 16)] = _dot16(buf, rvec, vc, brow)
            return carry
        lax.fori_loop(0, CHUNK // 16, g_body, 0)

    def flush_scores(ci, sbuf):
        @pl.when(ci < NB // CHUNK)
        def _():
            pltpu.sync_copy(sbuf, pos_out.at[pl.ds(base + ci * CHUNK, CHUNK)])

        @pl.when(ci >= NB // CHUNK)
        def _():
            pltpu.sync_copy(
                sbuf, neg_out.at[pl.ds(nbase + (ci - NB // CHUNK) * CHUNK, CHUNK)])

    def round_body(r, carry):
        for b in range(NBUF):
            ci = r * NBUF + b

            @pl.when(ci < NCHUNK)
            def _():
                wait_chunk(bufs[b], sems[b])

                @pl.when(ci + NBUF - 1 < NCHUNK)
                def _():
                    start_chunk(ci + NBUF - 1, bufs[b - 1], sems[b - 1])
                compute_chunk(ci, bufs[b], chunk_s)
                flush_scores(ci, chunk_s)
        return carry

    lax.fori_loop(0, NROUND, round_body, 0)


def _sc_entry(center_hbm, context_hbm, negflat_hbm, tbl_hbm,
              pos_out, neg_out,
              idx_c, idx_k, vc, b0, b1, b2, b3, b4, chunk_s,
              s0, s1, s2, s3, s4, semP):
    _sc_body(center_hbm, context_hbm, negflat_hbm, tbl_hbm,
             pos_out, neg_out, idx_c, idx_k, vc,
             [b0, b1, b2, b3, b4], chunk_s,
             [s0, s1, s2, s3, s4], semP)


_sc_scores = pl.kernel(
    _sc_entry,
    out_type=(jax.ShapeDtypeStruct((B,), jnp.float32),
              jax.ShapeDtypeStruct((B * NEG,), jnp.float32)),
    mesh=plsc.VectorSubcoreMesh(core_axis_name="c", subcore_axis_name="s"),
    scratch_types=[
        pltpu.VMEM((NB,), jnp.int32),
        pltpu.VMEM((NLOOK,), jnp.int32),
        pltpu.VMEM((NB, DIM), jnp.float32),
        pltpu.VMEM((CHUNK, DIM), jnp.float32),
        pltpu.VMEM((CHUNK, DIM), jnp.float32),
        pltpu.VMEM((CHUNK, DIM), jnp.float32),
        pltpu.VMEM((CHUNK, DIM), jnp.float32),
        pltpu.VMEM((CHUNK, DIM), jnp.float32),
        pltpu.VMEM((CHUNK,), jnp.float32),
        pltpu.SemaphoreType.DMA,
        pltpu.SemaphoreType.DMA,
        pltpu.SemaphoreType.DMA,
        pltpu.SemaphoreType.DMA,
        pltpu.SemaphoreType.DMA,
        pltpu.SemaphoreType.DMA,
    ],
    compiler_params=pltpu.CompilerParams(needs_layout_passes=False,
                                         use_tc_tiling_on_sc=False),
)


VOCAB = 1000000
TRB = 1024


def _tr_body(c_ref, x_ref, out_ref):
    out_ref[:, 0:DIM] = c_ref[...].T
    out_ref[:, DIM:2 * DIM] = x_ref[...].T


_tc_transpose = pl.pallas_call(
    _tr_body,
    grid=((VOCAB + TRB - 1) // TRB,),
    in_specs=[pl.BlockSpec((DIM, TRB), lambda i: (0, i)),
              pl.BlockSpec((DIM, TRB), lambda i: (0, i))],
    out_specs=pl.BlockSpec((TRB, 128), lambda i: (i, 0)),
    out_shape=jax.ShapeDtypeStruct((VOCAB, 128), jnp.float32),
)


def _to_table(embc, embx):
    return _tc_transpose(embc.T, embx.T).reshape(2 * VOCAB, DIM)


def _logsig(x):
    return jnp.minimum(x, 0.0) - jnp.log1p(jnp.exp(-jnp.abs(x)))


def _loss_body(pos_ref, neg_ref, out_ref):
    loss = -jnp.sum(_logsig(pos_ref[...])) - jnp.sum(_logsig(-neg_ref[...]))
    out_ref[0, 0] = loss


_tc_loss = pl.pallas_call(
    _loss_body,
    out_shape=jax.ShapeDtypeStruct((1, 1), jnp.float32),
    out_specs=pl.BlockSpec(memory_space=pltpu.SMEM),
)


def kernel(center_word, context_word, negative_samples, emb_center, emb_context):
    neg_flat = negative_samples.reshape(-1)
    pos_s, neg_s = _sc_scores(center_word, context_word, neg_flat,
                              _to_table(emb_center, emb_context))
    loss = _tc_loss(pos_s.reshape(B // 128, 128), neg_s.reshape(B * NEG // 128, 128))
    return loss[0, 0]
